# baseline (speedup 1.0000x reference)
import functools

import jax
import jax.numpy as jnp
from jax import lax
from jax.experimental import pallas as pl
from jax.experimental.pallas import tpu as pltpu
from jax.experimental.pallas import tpu_sc as plsc

N = 10000
E = 320000
D_IN = 128
D_HID = 16
D_OUT = 40
D_OUT_P = 48

NC = 2
NS = 16
NW = NC * NS

CH = 128
SUB = 80
PH = 2
SUBP = SUB // PH
SUB_A = 160
SUB_B = 0
PH_A = 4
PH_B = 1
SUBP_MAX = max(SUB_A // PH_A, SUB_B // PH_B)
NCHUNK = NS * (SUB_A + SUB_B)
TEP = SUB * CH
EP = NW * TEP
DUMMY = N

ACC_ROWS = 10112
STRIPE = 632

_mesh = plsc.VectorSubcoreMesh(core_axis_name="c", subcore_axis_name="s")


DEG_D = 16


@functools.partial(
    pl.kernel,
    mesh=_mesh,
    out_type=jax.ShapeDtypeStruct((NC, ACC_ROWS, DEG_D), jnp.float32),
    scratch_types=[
        pltpu.VMEM((SUB, CH), jnp.int32),
        pltpu.VMEM((CH, DEG_D), jnp.float32),
        pltpu.VMEM_SHARED((ACC_ROWS, DEG_D), jnp.float32),
    ],
)
def _deg_kernel(dst_hbm, out_hbm, dst_v, val_v, acc_sh):
    c = lax.axis_index("c")
    s = lax.axis_index("s")
    cb = (c * NS + s) * SUB

    for i in range(CH):
        val_v[i, :] = jnp.zeros((16,), jnp.float32)
    base = s * STRIPE
    for k in range(STRIPE // CH):
        pltpu.sync_copy(val_v, acc_sh.at[pl.ds(base + k * CH, CH)])
    pltpu.sync_copy(val_v.at[pl.ds(0, STRIPE % CH)],
                    acc_sh.at[pl.ds(base + (STRIPE // CH) * CH, STRIPE % CH)])

    for i in range(CH):
        val_v[i, :] = jnp.ones((16,), jnp.float32)
    pltpu.sync_copy(dst_hbm.at[pl.ds(cb, SUB)], dst_v)
    plsc.subcore_barrier()

    def body(j, carry):
        pltpu.sync_copy(val_v, acc_sh.at[dst_v.at[j]], add=True)
        return carry

    lax.fori_loop(0, SUB, body, 0)
    plsc.subcore_barrier()

    pltpu.sync_copy(acc_sh.at[pl.ds(s * STRIPE, STRIPE)],
                    out_hbm.at[c, pl.ds(s * STRIPE, STRIPE)])


@functools.partial(
    pl.kernel,
    mesh=_mesh,
    out_type=jax.ShapeDtypeStruct((NC, ACC_ROWS, 128), jnp.float32),
    scratch_types=[
        pltpu.VMEM((SUBP_MAX, CH), jnp.int32),
        pltpu.VMEM((SUBP_MAX, CH), jnp.int32),
        pltpu.VMEM((CH, 128), jnp.float32),
        pltpu.VMEM((CH, 128), jnp.float32),
        pltpu.VMEM_SHARED((ACC_ROWS, 128), jnp.float32),
        pltpu.SemaphoreType.DMA,
        pltpu.SemaphoreType.DMA,
    ],
)
def _agg(h_hbm, src_hbm, dst_hbm, out_hbm, src_v, dst_v, rows0_v, rows1_v,
         acc_sh, sem0, sem1):
    c = lax.axis_index("c")
    s = lax.axis_index("s")

    for i in range(CH):
        for k in range(8):
            rows0_v[i, pl.ds(k * 16, 16)] = jnp.zeros((16,), jnp.float32)
    base = s * STRIPE
    for k in range(STRIPE // CH):
        pltpu.sync_copy(rows0_v, acc_sh.at[pl.ds(base + k * CH, CH)])
    pltpu.sync_copy(rows0_v.at[pl.ds(0, STRIPE % CH)],
                    acc_sh.at[pl.ds(base + (STRIPE // CH) * CH, STRIPE % CH)])
    plsc.subcore_barrier()

    def run(cbase, nsub, ph):
        subp = nsub // ph
        for p in range(ph):
            pltpu.sync_copy(src_hbm.at[pl.ds(cbase + p * subp, subp)],
                            src_v.at[pl.ds(0, subp)])
            pltpu.sync_copy(dst_hbm.at[pl.ds(cbase + p * subp, subp)],
                            dst_v.at[pl.ds(0, subp)])
            pltpu.async_copy(h_hbm.at[src_v.at[0]], rows0_v, sem0)

            def body(jj, carry):
                j0 = 2 * jj
                pltpu.make_async_copy(h_hbm.at[src_v.at[0]], rows0_v,
                                      sem0).wait()
                pltpu.async_copy(h_hbm.at[src_v.at[j0 + 1]], rows1_v, sem1)
                pltpu.sync_copy(rows0_v, acc_sh.at[dst_v.at[j0]], add=True)
                pltpu.make_async_copy(h_hbm.at[src_v.at[0]], rows1_v,
                                      sem1).wait()
                nxt = jnp.minimum(j0 + 2, subp - 1)
                pltpu.async_copy(h_hbm.at[src_v.at[nxt]], rows0_v, sem0)
                pltpu.sync_copy(rows1_v, acc_sh.at[dst_v.at[j0 + 1]], add=True)
                return carry

            lax.fori_loop(0, subp // 2, body, 0)
            pltpu.make_async_copy(h_hbm.at[src_v.at[0]], rows0_v, sem0).wait()

    @pl.when(c == 0)
    def _():
        run(s * SUB_A, SUB_A, PH_A)

    if SUB_B:
        @pl.when(c == 1)
        def _():
            run(NS * SUB_A + s * SUB_B, SUB_B, PH_B)

    plsc.subcore_barrier()

    pltpu.sync_copy(acc_sh.at[pl.ds(s * STRIPE, STRIPE)],
                    out_hbm.at[c, pl.ds(s * STRIPE, STRIPE)])


_BLK = 1000
_GRID = N // _BLK


def _tc1_body(x_ref, w_ref, b_ref, deg_ref, o_ref):
    dis = lax.rsqrt(deg_ref[0, 0, :] + deg_ref[0, 1, :] + 1.0)
    h = jnp.dot(x_ref[...], w_ref[...], preferred_element_type=jnp.float32)
    o_ref[...] = (h + b_ref[...]) * dis[:, None]


def _tc2_body(agg_ref, h1s_ref, deg_ref, w_ref, b_ref, o_ref):
    dis = lax.rsqrt(deg_ref[0, 0, :] + deg_ref[0, 1, :] + 1.0)
    a = agg_ref[0, :, :D_HID] + agg_ref[1, :, :D_HID] + h1s_ref[:, :D_HID]
    z = jnp.maximum(a * dis[:, None], 0.0)
    h2 = jnp.dot(z, w_ref[...], preferred_element_type=jnp.float32)
    o_ref[...] = (h2 + b_ref[...]) * dis[:, None]


def _tc3_body(agg_ref, h2s_ref, deg_ref, o_ref):
    dis = lax.rsqrt(deg_ref[0, 0, :] + deg_ref[0, 1, :] + 1.0)
    o = ((agg_ref[0, :, :D_OUT] + agg_ref[1, :, :D_OUT] + h2s_ref[:, :D_OUT])
         * dis[:, None])
    m = jnp.max(o, axis=1, keepdims=True)
    e = jnp.exp(o - m)
    o_ref[...] = (o - m) - jnp.log(jnp.sum(e, axis=1, keepdims=True))


def _row_spec(d):
    return pl.BlockSpec((_BLK, d), lambda i: (i, 0))


_DEG_SPEC = pl.BlockSpec((1, 2, _BLK), lambda i: (i, 0, 0))


def _tc1(x, W1p, b1p, deg):
    return pl.pallas_call(
        _tc1_body,
        grid=(_GRID,),
        in_specs=[
            _row_spec(D_IN),
            pl.BlockSpec((D_IN, 128), lambda i: (0, 0)),
            pl.BlockSpec((1, 128), lambda i: (0, 0)),
            _DEG_SPEC,
        ],
        out_specs=_row_spec(128),
        out_shape=jax.ShapeDtypeStruct((N, 128), jnp.float32),
    )(x, W1p, b1p, deg)


def _tc2(agg1, h1s, deg, W2p, b2p):
    return pl.pallas_call(
        _tc2_body,
        grid=(_GRID,),
        in_specs=[
            pl.BlockSpec((2, _BLK, 128), lambda i: (0, i, 0)),
            _row_spec(128),
            _DEG_SPEC,
            pl.BlockSpec((D_HID, 128), lambda i: (0, 0)),
            pl.BlockSpec((1, 128), lambda i: (0, 0)),
        ],
        out_specs=_row_spec(128),
        out_shape=jax.ShapeDtypeStruct((N, 128), jnp.float32),
    )(agg1, h1s, deg, W2p, b2p)


def _tc3(agg2, h2s, deg):
    return pl.pallas_call(
        _tc3_body,
        grid=(_GRID,),
        in_specs=[
            pl.BlockSpec((2, _BLK, 128), lambda i: (0, i, 0)),
            _row_spec(128),
            _DEG_SPEC,
        ],
        out_specs=_row_spec(D_OUT),
        out_shape=jax.ShapeDtypeStruct((N, D_OUT), jnp.float32),
    )(agg2, h2s, deg)


def kernel(x, edge_index, W1, b1, W2, b2):
    src = edge_index[0].astype(jnp.int32)
    dst = edge_index[1].astype(jnp.int32)
    pad = EP - E
    src_p = jnp.concatenate(
        [src, jnp.zeros((pad,), jnp.int32)]).reshape(NCHUNK, CH)
    dst_p = jnp.concatenate(
        [dst, jnp.full((pad,), DUMMY, jnp.int32)]).reshape(NCHUNK, CH)

    dego = _deg_kernel(dst_p)
    deg = dego[:, :N, 0].reshape(2, _GRID, _BLK).transpose(1, 0, 2)

    W1p = jnp.pad(W1, ((0, 0), (0, 128 - D_HID)))
    b1p = jnp.pad(b1, (0, 128 - D_HID)).reshape(1, 128)
    h1s = _tc1(x, W1p, b1p, deg)
    agg1 = _agg(h1s, src_p, dst_p)

    W2p = jnp.pad(W2, ((0, 0), (0, 128 - D_OUT)))
    b2p = jnp.pad(b2, (0, 128 - D_OUT)).reshape(1, 128)
    h2s = _tc2(agg1, h1s, deg, W2p, b2p)
    agg2 = _agg(h2s, src_p, dst_p)

    return _tc3(agg2, h2s, deg)


# speedup vs baseline: 1.3983x; 1.3983x over previous
import functools

import jax
import jax.numpy as jnp
from jax import lax
from jax.experimental import pallas as pl
from jax.experimental.pallas import tpu as pltpu
from jax.experimental.pallas import tpu_sc as plsc

N = 10000
E = 320000
D_IN = 128
D_HID = 16
D_OUT = 40
D_OUT_P = 48

NC = 2
NS = 16
NW = NC * NS

CH = 128
SUB = 80
PH = 2
SUBP = SUB // PH
SUB_A = 144
SUB_B = 16
PH_A = 3
PH_B = 1
SUBP_MAX = max(SUB_A // PH_A, SUB_B // PH_B)
NCHUNK = NS * (SUB_A + SUB_B)
TEP = SUB * CH
EP = NW * TEP
DUMMY = N

ACC_ROWS = 10112
STRIPE = 632

_mesh = plsc.VectorSubcoreMesh(core_axis_name="c", subcore_axis_name="s")


DEG_D = 16


@functools.partial(
    pl.kernel,
    mesh=_mesh,
    out_type=jax.ShapeDtypeStruct((NC, ACC_ROWS, DEG_D), jnp.float32),
    scratch_types=[
        pltpu.VMEM((SUB, CH), jnp.int32),
        pltpu.VMEM((CH, DEG_D), jnp.float32),
        pltpu.VMEM_SHARED((ACC_ROWS, DEG_D), jnp.float32),
    ],
)
def _deg_kernel(dst_hbm, out_hbm, dst_v, val_v, acc_sh):
    c = lax.axis_index("c")
    s = lax.axis_index("s")
    cb = (c * NS + s) * SUB

    for i in range(CH):
        val_v[i, :] = jnp.zeros((16,), jnp.float32)
    base = s * STRIPE
    for k in range(STRIPE // CH):
        pltpu.sync_copy(val_v, acc_sh.at[pl.ds(base + k * CH, CH)])
    pltpu.sync_copy(val_v.at[pl.ds(0, STRIPE % CH)],
                    acc_sh.at[pl.ds(base + (STRIPE // CH) * CH, STRIPE % CH)])

    for i in range(CH):
        val_v[i, :] = jnp.ones((16,), jnp.float32)
    pltpu.sync_copy(dst_hbm.at[pl.ds(cb, SUB)], dst_v)
    plsc.subcore_barrier()

    def body(j, carry):
        pltpu.sync_copy(val_v, acc_sh.at[dst_v.at[j]], add=True)
        return carry

    lax.fori_loop(0, SUB, body, 0)
    plsc.subcore_barrier()

    pltpu.sync_copy(acc_sh.at[pl.ds(s * STRIPE, STRIPE)],
                    out_hbm.at[c, pl.ds(s * STRIPE, STRIPE)])


@functools.partial(
    pl.kernel,
    mesh=_mesh,
    out_type=jax.ShapeDtypeStruct((NC, ACC_ROWS, 128), jnp.float32),
    scratch_types=[
        pltpu.VMEM((SUBP_MAX, CH), jnp.int32),
        pltpu.VMEM((SUBP_MAX, CH), jnp.int32),
        pltpu.VMEM((CH, 128), jnp.float32),
        pltpu.VMEM((CH, 128), jnp.float32),
        pltpu.VMEM_SHARED((ACC_ROWS, 128), jnp.float32),
        pltpu.SemaphoreType.DMA,
        pltpu.SemaphoreType.DMA,
        pltpu.SemaphoreType.DMA,
        pltpu.SemaphoreType.DMA,
    ],
)
def _agg(h_hbm, src_hbm, dst_hbm, out_hbm, src_v, dst_v, rows0_v, rows1_v,
         acc_sh, sem0, sem1, sems0, sems1):
    c = lax.axis_index("c")
    s = lax.axis_index("s")

    for i in range(CH):
        for k in range(8):
            rows0_v[i, pl.ds(k * 16, 16)] = jnp.zeros((16,), jnp.float32)
    base = s * STRIPE
    for k in range(STRIPE // CH):
        pltpu.sync_copy(rows0_v, acc_sh.at[pl.ds(base + k * CH, CH)])
    pltpu.sync_copy(rows0_v.at[pl.ds(0, STRIPE % CH)],
                    acc_sh.at[pl.ds(base + (STRIPE // CH) * CH, STRIPE % CH)])
    plsc.subcore_barrier()

    def run(cbase, nsub, ph):
        subp = nsub // ph
        for p in range(ph):
            pltpu.sync_copy(src_hbm.at[pl.ds(cbase + p * subp, subp)],
                            src_v.at[pl.ds(0, subp)])
            pltpu.sync_copy(dst_hbm.at[pl.ds(cbase + p * subp, subp)],
                            dst_v.at[pl.ds(0, subp)])
            pltpu.async_copy(h_hbm.at[src_v.at[0]], rows0_v, sem0)

            def body(jj, carry):
                j0 = 2 * jj
                pltpu.make_async_copy(h_hbm.at[src_v.at[0]], rows0_v,
                                      sem0).wait()
                pltpu.async_copy(h_hbm.at[src_v.at[j0 + 1]], rows1_v, sem1)
                sc0 = pltpu.async_copy(rows0_v, acc_sh.at[dst_v.at[j0]],
                                       sems0, add=True)
                pltpu.make_async_copy(h_hbm.at[src_v.at[0]], rows1_v,
                                      sem1).wait()
                sc0.wait()
                nxt = jnp.minimum(j0 + 2, subp - 1)
                pltpu.async_copy(h_hbm.at[src_v.at[nxt]], rows0_v, sem0)
                sc1 = pltpu.async_copy(rows1_v, acc_sh.at[dst_v.at[j0 + 1]],
                                       sems1, add=True)
                sc1.wait()
                return carry

            lax.fori_loop(0, subp // 2, body, 0)
            pltpu.make_async_copy(h_hbm.at[src_v.at[0]], rows0_v, sem0).wait()

    @pl.when(c == 0)
    def _():
        run(s * SUB_A, SUB_A, PH_A)

    if SUB_B:
        @pl.when(c == 1)
        def _():
            run(NS * SUB_A + s * SUB_B, SUB_B, PH_B)

    plsc.subcore_barrier()

    pltpu.sync_copy(acc_sh.at[pl.ds(s * STRIPE, STRIPE)],
                    out_hbm.at[c, pl.ds(s * STRIPE, STRIPE)])


_BLK = 1000
_GRID = N // _BLK


def _tc1_body(x_ref, w_ref, b_ref, deg_ref, o_ref):
    dis = lax.rsqrt(deg_ref[0, 0, :] + deg_ref[0, 1, :] + 1.0)
    h = jnp.dot(x_ref[...], w_ref[...], preferred_element_type=jnp.float32)
    o_ref[...] = (h + b_ref[...]) * dis[:, None]


def _tc2_body(agg_ref, h1s_ref, deg_ref, w_ref, b_ref, o_ref):
    dis = lax.rsqrt(deg_ref[0, 0, :] + deg_ref[0, 1, :] + 1.0)
    a = agg_ref[0, :, :D_HID] + agg_ref[1, :, :D_HID] + h1s_ref[:, :D_HID]
    z = jnp.maximum(a * dis[:, None], 0.0)
    h2 = jnp.dot(z, w_ref[...], preferred_element_type=jnp.float32)
    o_ref[...] = (h2 + b_ref[...]) * dis[:, None]


def _tc3_body(agg_ref, h2s_ref, deg_ref, o_ref):
    dis = lax.rsqrt(deg_ref[0, 0, :] + deg_ref[0, 1, :] + 1.0)
    o = ((agg_ref[0, :, :D_OUT] + agg_ref[1, :, :D_OUT] + h2s_ref[:, :D_OUT])
         * dis[:, None])
    m = jnp.max(o, axis=1, keepdims=True)
    e = jnp.exp(o - m)
    o_ref[...] = (o - m) - jnp.log(jnp.sum(e, axis=1, keepdims=True))


def _row_spec(d):
    return pl.BlockSpec((_BLK, d), lambda i: (i, 0))


_DEG_SPEC = pl.BlockSpec((1, 2, _BLK), lambda i: (i, 0, 0))


def _tc1(x, W1p, b1p, deg):
    return pl.pallas_call(
        _tc1_body,
        grid=(_GRID,),
        in_specs=[
            _row_spec(D_IN),
            pl.BlockSpec((D_IN, 128), lambda i: (0, 0)),
            pl.BlockSpec((1, 128), lambda i: (0, 0)),
            _DEG_SPEC,
        ],
        out_specs=_row_spec(128),
        out_shape=jax.ShapeDtypeStruct((N, 128), jnp.float32),
    )(x, W1p, b1p, deg)


def _tc2(agg1, h1s, deg, W2p, b2p):
    return pl.pallas_call(
        _tc2_body,
        grid=(_GRID,),
        in_specs=[
            pl.BlockSpec((2, _BLK, 128), lambda i: (0, i, 0)),
            _row_spec(128),
            _DEG_SPEC,
            pl.BlockSpec((D_HID, 128), lambda i: (0, 0)),
            pl.BlockSpec((1, 128), lambda i: (0, 0)),
        ],
        out_specs=_row_spec(128),
        out_shape=jax.ShapeDtypeStruct((N, 128), jnp.float32),
    )(agg1, h1s, deg, W2p, b2p)


def _tc3(agg2, h2s, deg):
    return pl.pallas_call(
        _tc3_body,
        grid=(_GRID,),
        in_specs=[
            pl.BlockSpec((2, _BLK, 128), lambda i: (0, i, 0)),
            _row_spec(128),
            _DEG_SPEC,
        ],
        out_specs=_row_spec(D_OUT),
        out_shape=jax.ShapeDtypeStruct((N, D_OUT), jnp.float32),
    )(agg2, h2s, deg)


def kernel(x, edge_index, W1, b1, W2, b2):
    src = edge_index[0].astype(jnp.int32)
    dst = edge_index[1].astype(jnp.int32)
    pad = EP - E
    src_p = jnp.concatenate(
        [src, jnp.zeros((pad,), jnp.int32)]).reshape(NCHUNK, CH)
    dst_p = jnp.concatenate(
        [dst, jnp.full((pad,), DUMMY, jnp.int32)]).reshape(NCHUNK, CH)

    dego = _deg_kernel(dst_p)
    deg = dego[:, :N, 0].reshape(2, _GRID, _BLK).transpose(1, 0, 2)

    W1p = jnp.pad(W1, ((0, 0), (0, 128 - D_HID)))
    b1p = jnp.pad(b1, (0, 128 - D_HID)).reshape(1, 128)
    h1s = _tc1(x, W1p, b1p, deg)
    agg1 = _agg(h1s, src_p, dst_p)

    W2p = jnp.pad(W2, ((0, 0), (0, 128 - D_OUT)))
    b2p = jnp.pad(b2, (0, 128 - D_OUT)).reshape(1, 128)
    h2s = _tc2(agg1, h1s, deg, W2p, b2p)
    agg2 = _agg(h2s, src_p, dst_p)

    return _tc3(agg2, h2s, deg)
